# baseline (speedup 1.0000x reference)
import jax
import jax.numpy as jnp
from jax.experimental import pallas as pl

BM = 256


def _agg_kernel(user_ref, latent_ref, weight_ref, interact_ref, entity_ref,
                out_ref):
    agg = jnp.dot(interact_ref[...].astype(jnp.bfloat16),
                  entity_ref[...].astype(jnp.bfloat16),
                  preferred_element_type=jnp.float32)
    score = jnp.dot(user_ref[...], latent_ref[...].T,
                    preferred_element_type=jnp.float32)
    score = jax.nn.softmax(score, axis=1)
    gate = jnp.dot(score, weight_ref[...],
                   preferred_element_type=jnp.float32)
    out_ref[...] = agg * (1.0 + gate)


@jax.jit
def kernel(entity_emb, user_emb, latent_emb, weight, interact_mat):
    n_users, n_entities = interact_mat.shape
    channel = entity_emb.shape[1]
    nm = n_users // BM

    return pl.pallas_call(
        _agg_kernel,
        grid=(nm,),
        in_specs=[
            pl.BlockSpec((BM, channel), lambda m: (m, 0)),
            pl.BlockSpec(latent_emb.shape, lambda m: (0, 0)),
            pl.BlockSpec(weight.shape, lambda m: (0, 0)),
            pl.BlockSpec((BM, n_entities), lambda m: (m, 0)),
            pl.BlockSpec((n_entities, channel), lambda m: (0, 0)),
        ],
        out_specs=pl.BlockSpec((BM, channel), lambda m: (m, 0)),
        out_shape=jax.ShapeDtypeStruct((n_users, channel), jnp.float32),
    )(user_emb, latent_emb, weight, interact_mat, entity_emb)
